# Initial kernel scaffold; baseline (speedup 1.0000x reference)
import jax
import jax.numpy as jnp
from jax.experimental import pallas as pl


def kernel(src, edge_index, dst, segment_ids, W, b):
    raise NotImplementedError("write your pallas kernel here")


# speedup vs baseline: 2.2819x; 2.2819x over previous
import functools

import jax
import jax.numpy as jnp
from jax import lax
from jax.experimental import pallas as pl
from jax.experimental.pallas import tpu as pltpu
from jax.experimental.pallas import tpu_sc as plsc

_NC = 2
_NS = 16


def _sc_segment_acc(dst, segment_ids, n_nodes, width):
    e = segment_ids.shape[0]
    nw = _NC * _NS
    epw = e // nw
    c = 40
    iters = epw // c
    wrows = (n_nodes // (8 * _NS)) * 8
    wchunk = 24
    tail = n_nodes - _NS * wrows
    zc_stripe = -(-n_nodes // (_NS * 16)) * 16
    zc_last = n_nodes - (_NS - 1) * zc_stripe
    assert zc_last > 0 and zc_last % 16 == 0
    with_rows = dst is not None

    mesh = plsc.VectorSubcoreMesh(core_axis_name="c", subcore_axis_name="s")

    @functools.partial(
        pl.kernel,
        mesh=mesh,
        out_type=jax.ShapeDtypeStruct((_NC, n_nodes, width), jnp.float32),
        scratch_types=[
            pltpu.VMEM((c,), jnp.int32),
            pltpu.VMEM((c, width), jnp.float32),
            pltpu.VMEM((16, width), jnp.float32),
            pltpu.VMEM((16,), jnp.int32),
            pltpu.VMEM((wchunk, width), jnp.float32),
            pltpu.VMEM_SHARED((n_nodes, width), jnp.float32),
        ],
    )
    def k(*refs):
        if with_rows:
            (dst_hbm, seg_hbm, out_hbm,
             idx_v, rows_v, zrows_v, idx16_v, bnc_v, acc_sh) = refs
        else:
            (seg_hbm, out_hbm,
             idx_v, rows_v, zrows_v, idx16_v, bnc_v, acc_sh) = refs
        cid = lax.axis_index("c")
        sid = lax.axis_index("s")
        wid = cid * _NS + sid
        iota = lax.iota(jnp.int32, 16)

        def fill_z(i, carry):
            zrows_v[i // (width // 16), pl.ds((i % (width // 16)) * 16, 16)] \
                = jnp.zeros((16,), jnp.float32)
            return carry
        lax.fori_loop(0, 16 * (width // 16), fill_z, 0)

        if not with_rows:
            def fill_one(i, carry):
                rows_v[i // (width // 16),
                       pl.ds((i % (width // 16)) * 16, 16)] = (
                    jnp.ones((16,), jnp.float32))
                return carry
            lax.fori_loop(0, c * (width // 16), fill_one, 0)

        def zero_op(j, carry):
            idx16_v[...] = sid * zc_stripe + j * 16 + iota
            pltpu.sync_copy(zrows_v, acc_sh.at[idx16_v])
            return carry
        nops = lax.select(sid == _NS - 1, zc_last // 16, zc_stripe // 16)
        lax.fori_loop(0, nops, zero_op, 0)
        plsc.subcore_barrier()

        ebase = wid * epw

        def step(i, carry):
            base = ebase + i * c
            pltpu.sync_copy(seg_hbm.at[pl.ds(base, c)], idx_v)
            if with_rows:
                pltpu.sync_copy(dst_hbm.at[pl.ds(base, c)], rows_v)
            pltpu.sync_copy(rows_v, acc_sh.at[idx_v], add=True)
            return carry
        lax.fori_loop(0, iters, step, 0)
        plsc.subcore_barrier()

        def wr_blk(j, carry):
            base = sid * wrows + j * wchunk
            pltpu.sync_copy(acc_sh.at[pl.ds(base, wchunk)], bnc_v)
            pltpu.sync_copy(bnc_v, out_hbm.at[cid, pl.ds(base, wchunk)])
            return carry
        lax.fori_loop(0, wrows // wchunk, wr_blk, 0)

        @pl.when(sid == _NS - 1)
        def _():
            tb = _NS * wrows
            pltpu.sync_copy(acc_sh.at[pl.ds(tb, tail)],
                            bnc_v.at[pl.ds(0, tail)])
            pltpu.sync_copy(bnc_v.at[pl.ds(0, tail)],
                            out_hbm.at[cid, pl.ds(tb, tail)])

    if with_rows:
        return k(dst, segment_ids)
    return k(segment_ids)


def _tc_finish(sums, cnts, src, w, b2d):
    n, d = src.shape
    u = w.shape[1]
    bn = 400

    def body(s_ref, c_ref, x_ref, w_ref, b_ref, o_ref):
        s = s_ref[0] + s_ref[1]
        cnt = c_ref[0, :, 0:1] + c_ref[1, :, 0:1]
        x = s / jnp.maximum(cnt, 1.0) + x_ref[...]
        o_ref[...] = (
            jnp.dot(x, w_ref[...], preferred_element_type=jnp.float32)
            + b_ref[...])

    return pl.pallas_call(
        body,
        grid=(n // bn,),
        in_specs=[
            pl.BlockSpec((_NC, bn, d), lambda i: (0, i, 0)),
            pl.BlockSpec((_NC, bn, d), lambda i: (0, i, 0)),
            pl.BlockSpec((bn, d), lambda i: (i, 0)),
            pl.BlockSpec((d, u), lambda i: (0, 0)),
            pl.BlockSpec((1, u), lambda i: (0, 0)),
        ],
        out_specs=pl.BlockSpec((bn, u), lambda i: (i, 0)),
        out_shape=jax.ShapeDtypeStruct((n, u), jnp.float32),
    )(sums, cnts, src, w, b2d)


def kernel(src, edge_index, dst, segment_ids, W, b):
    n = src.shape[0]
    sums = _sc_segment_acc(dst, segment_ids, n, dst.shape[1])
    cnts = _sc_segment_acc(None, segment_ids, n, dst.shape[1])
    return _tc_finish(sums, cnts, src, W, b.reshape(1, -1))
